# Initial kernel scaffold; baseline (speedup 1.0000x reference)
import jax
import jax.numpy as jnp
from jax.experimental import pallas as pl


def kernel(node_states, edge_states, scalars, batched_reverse_idx, edge_index, batch_scalars, emb_node, emb_edge, W_combine, b_combine, W_keep, b_keep, W_push, b_push, W_push_node, b_push_node, W_incr, b_incr, training_step, processor_step, teacher_force):
    raise NotImplementedError("write your pallas kernel here")


# speedup vs baseline: 32.8109x; 32.8109x over previous
import functools

import jax
import jax.numpy as jnp
from jax import lax
from jax.experimental import pallas as pl
from jax.experimental.pallas import tpu as pltpu
from jax.experimental.pallas import tpu_sc as plsc

NC = 2
NS = 16
NW = NC * NS
L = 16

_f32 = jnp.float32
_i32 = jnp.int32


def _tables_body(ee, en, w1, w2, vmat, c0row, bcrow, tinv, de_o, dn_o):
    vs = vmat[...] * tinv[0, 0]
    u1 = jnp.dot(w1[...], vs, preferred_element_type=_f32)
    u2 = jnp.dot(w2[...], vs, preferred_element_type=_f32)
    de_o[...] = jnp.dot(ee[...], u1, preferred_element_type=_f32)
    bias = jnp.dot(bcrow[...], vs, preferred_element_type=_f32) + c0row[...] * tinv[0, 0]
    dn_o[...] = jnp.dot(en[...], u2, preferred_element_type=_f32) + bias


def _tables_call(ee, en, w1, w2, vmat, c0row, bcrow, tinv):
    n_codes = ee.shape[0]
    return pl.pallas_call(
        _tables_body,
        out_shape=(
            jax.ShapeDtypeStruct((n_codes, 128), _f32),
            jax.ShapeDtypeStruct((n_codes, 128), _f32),
        ),
    )(ee, en, w1, w2, vmat, c0row, bcrow, tinv)


def _edge_body(E_RAND, Q, K, NP,
               ecode_h, brev_h, src_h, dst_h, s_h, bs_h, de_h, dn_h,
               ncode_h, ns_h, zeros_h,
               out_h, accs_h, lp_h,
               brev_v, acode_v, src_v, dst_v, s_v, bs_v, out_v, val_v,
               de_v, dn_v, ncode_v, ns_v, lsum_v, acc_sh, gsem, ssem):
    cid = lax.axis_index("c")
    sid = lax.axis_index("s")
    wid = cid * NS + sid

    @pl.when(sid == 0)
    def _():
        pltpu.sync_copy(zeros_h, acc_sh)

    plsc.subcore_barrier()

    pltpu.sync_copy(de_h, de_v)
    pltpu.sync_copy(dn_h, dn_v)
    pltpu.sync_copy(ncode_h, ncode_v)
    pltpu.sync_copy(ns_h, ns_v)
    pltpu.sync_copy(brev_h.at[wid], brev_v)
    pltpu.sync_copy(src_h.at[wid], src_v)
    pltpu.sync_copy(dst_h.at[wid], dst_v)
    pltpu.sync_copy(s_h.at[wid], s_v)
    pltpu.sync_copy(bs_h.at[wid], bs_v)

    def _gfire(j, c):
        pltpu.async_copy(ecode_h.at[brev_v.at[j]], acode_v.at[j], gsem)
        return c

    lax.fori_loop(0, K, _gfire, 0)

    def _gdrain(j, c):
        pltpu.make_async_copy(ecode_h.at[brev_v.at[j]], acode_v.at[j], gsem).wait()
        return c

    lax.fori_loop(0, K, _gdrain, 0)

    iota16 = lax.iota(_i32, L)

    def _jbody(j, lacc):
        for i in range(128 // L):
            sl = pl.ds(i * L, L)
            a = acode_v[j, sl]
            srcv = src_v[j, sl]
            dstv = dst_v[j, sl]
            sv = s_v[j, sl]
            bsv = bs_v[j, sl]
            b = plsc.load_gather(ncode_v, [srcv])
            nss = plsc.load_gather(ns_v, [srcv])
            nsd = plsc.load_gather(ns_v, [dstv])
            a4 = a * 4
            b4 = b * 4
            x_incr = plsc.load_gather(de_v, [a4]) + plsc.load_gather(dn_v, [b4])
            x_pwo = plsc.load_gather(de_v, [a4 + 1]) + plsc.load_gather(dn_v, [b4 + 1])
            x_pw = plsc.load_gather(de_v, [a4 + 2]) + plsc.load_gather(dn_v, [b4 + 2])
            x_keep = plsc.load_gather(de_v, [a4 + 3]) + plsc.load_gather(dn_v, [b4 + 3])
            incr = 1.0 / (1.0 + jnp.exp(-x_incr))
            pwo = 1.0 / (1.0 + jnp.exp(-x_pwo))
            pw = 1.0 / (1.0 + jnp.exp(-x_pw))
            keep = 1.0 / (1.0 + jnp.exp(-x_keep))
            swo = sv - nsd
            swn = swo + nss
            val = pwo * swo + pw * swn
            outv = incr + keep * sv
            out_v[j, sl] = outv
            val_v[j, sl] = val
            gidx = wid * Q + j * 128 + i * L + iota16
            d = bsv - outv
            lacc = lacc + jnp.where(gidx < E_RAND, d * d, 0.0)
        return lacc

    lacc = lax.fori_loop(0, K, _jbody, jnp.zeros((L,), _f32))
    lsum_v[...] = lacc

    pltpu.sync_copy(out_v, out_h.at[wid])

    def _sfire(j, c):
        pltpu.async_copy(val_v.at[j], acc_sh.at[dst_v.at[j]], ssem, add=True)
        return c

    lax.fori_loop(0, K, _sfire, 0)

    def _sdrain(j, c):
        pltpu.make_async_copy(val_v.at[j], acc_sh.at[dst_v.at[j]], ssem).wait()
        return c

    lax.fori_loop(0, K, _sdrain, 0)

    plsc.subcore_barrier()

    @pl.when(sid == 0)
    def _():
        pltpu.sync_copy(acc_sh, accs_h.at[pl.ds(cid * NP, NP)])

    pltpu.sync_copy(lsum_v, lp_h.at[wid])


def _edge_call(E_RAND, Q, K, NP,
               ecode, brev_p, src_p, dst_p, s_p, bs_p, detab, dntab, ncode_p,
               ns_p, zeros_np):
    mesh = plsc.VectorSubcoreMesh(core_axis_name="c", subcore_axis_name="s")
    kern = functools.partial(
        pl.kernel,
        out_type=(
            jax.ShapeDtypeStruct((NW, K, 128), _f32),
            jax.ShapeDtypeStruct((NC * NP,), _f32),
            jax.ShapeDtypeStruct((NW, L), _f32),
        ),
        mesh=mesh,
        scratch_types=[
            pltpu.VMEM((K, 128), _i32),
            pltpu.VMEM((K, 128), _i32),
            pltpu.VMEM((K, 128), _i32),
            pltpu.VMEM((K, 128), _i32),
            pltpu.VMEM((K, 128), _f32),
            pltpu.VMEM((K, 128), _f32),
            pltpu.VMEM((K, 128), _f32),
            pltpu.VMEM((K, 128), _f32),
            pltpu.VMEM((1024,), _f32),
            pltpu.VMEM((1024,), _f32),
            pltpu.VMEM((NP,), _i32),
            pltpu.VMEM((NP,), _f32),
            pltpu.VMEM((L,), _f32),
            pltpu.VMEM_SHARED((NP,), _f32),
            pltpu.SemaphoreType.DMA,
            pltpu.SemaphoreType.DMA,
        ],
        compiler_params=pltpu.CompilerParams(needs_layout_passes=False),
    )(functools.partial(_edge_body, E_RAND, Q, K, NP))
    return kern(ecode, brev_p, src_p, dst_p, s_p, bs_p, detab, dntab,
                ncode_p, ns_p, zeros_np)


def _node_body(N, PN, NP, accs_h, outt_h, bst_h, selfout_h, lp2_h,
               a0_v, a1_v, ot_v, bt_v, so_v, ls_v):
    wid = lax.axis_index("c") * NS + lax.axis_index("s")
    base = wid * PN
    pltpu.sync_copy(accs_h.at[pl.ds(base, PN)], a0_v)
    pltpu.sync_copy(accs_h.at[pl.ds(NP + base, PN)], a1_v)
    pltpu.sync_copy(outt_h.at[pl.ds(base, PN)], ot_v)
    pltpu.sync_copy(bst_h.at[pl.ds(base, PN)], bt_v)
    iota16 = lax.iota(_i32, L)
    lacc = jnp.zeros((L,), _f32)
    for i in range(PN // L):
        sl = pl.ds(i * L, L)
        so = ot_v[sl] + a0_v[sl] + a1_v[sl]
        so_v[sl] = so
        nid = base + i * L + iota16
        d = bt_v[sl] - so
        lacc = lacc + jnp.where(nid < N, d * d, 0.0)
    ls_v[...] = lacc
    pltpu.sync_copy(so_v, selfout_h.at[pl.ds(base, PN)])
    pltpu.sync_copy(ls_v, lp2_h.at[wid])


def _node_call(N, NP, accs, outt, bst):
    PN = NP // NW
    mesh = plsc.VectorSubcoreMesh(core_axis_name="c", subcore_axis_name="s")
    kern = functools.partial(
        pl.kernel,
        out_type=(
            jax.ShapeDtypeStruct((NP,), _f32),
            jax.ShapeDtypeStruct((NW, L), _f32),
        ),
        mesh=mesh,
        scratch_types=[
            pltpu.VMEM((PN,), _f32),
            pltpu.VMEM((PN,), _f32),
            pltpu.VMEM((PN,), _f32),
            pltpu.VMEM((PN,), _f32),
            pltpu.VMEM((PN,), _f32),
            pltpu.VMEM((L,), _f32),
        ],
        compiler_params=pltpu.CompilerParams(needs_layout_passes=False),
    )(functools.partial(_node_body, N, PN, NP))
    return kern(accs, outt, bst)


def _pad_to(x, n, val):
    return jnp.concatenate(
        [x, jnp.full((n - x.shape[0],), val, x.dtype)]) if n > x.shape[0] else x


def kernel(node_states, edge_states, scalars, batched_reverse_idx, edge_index,
           batch_scalars, emb_node, emb_edge, W_combine, b_combine,
           W_keep, b_keep, W_push, b_push, W_push_node, b_push_node,
           W_incr, b_incr, training_step, processor_step, teacher_force):
    N = node_states.shape[0]
    E = scalars.shape[0]
    E_RAND = E - N
    H = emb_node.shape[1]

    K = -(-E // (NW * 128))
    Q = K * 128
    EPAD = NW * Q
    PN = -(-N // (NW * L)) * L
    NP = PN * NW
    while EPAD < E_RAND + NP:
        K += 1
        Q = K * 128
        EPAD = NW * Q

    pn2 = 2 ** jnp.arange(node_states.shape[1], dtype=_i32)
    pe2 = 2 ** jnp.arange(edge_states.shape[1], dtype=_i32)
    ncode = jnp.sum(node_states * pn2[None, :], axis=1).astype(_i32)
    ecode = jnp.sum(edge_states * pe2[None, :], axis=1).astype(_i32)

    ts = jnp.asarray(training_step)
    frac = jnp.minimum(ts.astype(_f32) / 10000.0, 1.0)
    tau = jnp.where(ts == -1, 0.5, 3.0 + frac * (0.5 - 3.0))
    tinv = (1.0 / tau).reshape(1, 1).astype(_f32)

    Vmat = jnp.stack([
        W_incr[:, 0] - W_incr[:, 1],
        W_push[:, 0] - W_push[:, 1],
        W_push_node[:, 0] - W_push_node[:, 1],
        W_keep[:, 0] - W_keep[:, 1],
    ], axis=1)
    Vpad = jnp.zeros((H, 128), _f32).at[:, :4].set(Vmat)
    c0 = jnp.stack([
        b_incr[0] - b_incr[1],
        b_push[0] - b_push[1],
        b_push_node[0] - b_push_node[1],
        b_keep[0] - b_keep[1],
    ])
    c0row = jnp.zeros((1, 128), _f32).at[0, :4].set(c0)
    bcrow = b_combine.reshape(1, H)

    de2d, dn2d = _tables_call(emb_edge, emb_node, W_combine[:H], W_combine[H:],
                              Vpad, c0row, bcrow, tinv)
    detab = de2d[:, :4].reshape(-1)
    dntab = dn2d[:, :4].reshape(-1)

    s_flat = scalars.reshape(-1)
    src = edge_index[0]
    dst = edge_index[1]
    bsf = lax.dynamic_slice_in_dim(batch_scalars[:, :, 0], processor_step, 1,
                                   axis=1).reshape(-1)
    ns_node = s_flat[E_RAND:]

    def lay(x):
        return x.reshape(NW, K, 128)

    brev_p = lay(_pad_to(batched_reverse_idx, EPAD, 0))
    src_p = lay(_pad_to(src, EPAD, 0))
    dst_p = lay(_pad_to(dst, EPAD, NP - 1))
    s_p = lay(_pad_to(s_flat, EPAD, 0.0))
    bs_p = lay(_pad_to(bsf, EPAD, 0.0))
    ncode_p = _pad_to(ncode, NP, 0)
    ns_p = _pad_to(ns_node, NP, 0.0)
    zeros_np = jnp.zeros((NP,), _f32)

    out4, accs, lp1 = _edge_call(E_RAND, Q, K, NP, ecode, brev_p, src_p, dst_p,
                                 s_p, bs_p, detab, dntab, ncode_p, ns_p,
                                 zeros_np)

    out_flat = out4.reshape(EPAD)
    outt = out_flat[E_RAND:E_RAND + NP]
    bst = _pad_to(bsf[E_RAND:], NP, 0.0)
    selfout, lp2 = _node_call(N, NP, accs, outt, bst)

    new_flat = jnp.concatenate([out_flat[:E_RAND], selfout[:N]])
    loss = (jnp.sum(lp1) + jnp.sum(lp2)) / E
    new_scalars = new_flat[:, None]
    new_scalars = jnp.where(teacher_force, bsf[:, None], new_scalars)
    return (new_scalars, loss)
